# initial kernel scaffold (unmeasured)
import jax
import jax.numpy as jnp
from jax import lax
from jax.experimental import pallas as pl
from jax.experimental.pallas import tpu as pltpu

N_DEV = 4
EPS = 1e-5


def kernel(x, gamma):
    m, n = x.shape
    n_global = N_DEV * n
    gamma2d = gamma.reshape(1, n)
    sub, lane = 8, 128
    assert sub * lane == m

    def body(x_ref, g_ref, out_ref, comm_ref, send_sems, recv_sems):
        my_pos = lax.axis_index("i")
        left = jax.lax.rem(my_pos + N_DEV - 1, N_DEV)
        right = jax.lax.rem(my_pos + 1, N_DEV)

        barrier_sem = pltpu.get_barrier_semaphore()
        for nbr in [left, right]:
            pl.semaphore_signal(
                barrier_sem, inc=1,
                device_id=(nbr,), device_id_type=pl.DeviceIdType.MESH,
            )
        pl.semaphore_wait(barrier_sem, 2)

        xv = x_ref[:, :]
        partial = jnp.sum(xv * xv, axis=1)
        comm_ref[0, :, :] = partial.reshape(sub, lane)

        for h in range(N_DEV - 1):
            rdma = pltpu.make_async_remote_copy(
                src_ref=comm_ref.at[h],
                dst_ref=comm_ref.at[h + 1],
                send_sem=send_sems.at[h],
                recv_sem=recv_sems.at[h],
                device_id=(right,),
                device_id_type=pl.DeviceIdType.MESH,
            )
            rdma.start()
            rdma.wait()

        total = (
            comm_ref[0, :, :] + comm_ref[1, :, :]
            + comm_ref[2, :, :] + comm_ref[3, :, :]
        )
        inv_rms = lax.rsqrt(total.reshape(m, 1) / n_global + EPS)
        out_ref[:, :] = xv * g_ref[0, :][None, :] * inv_rms

    return pl.pallas_call(
        body,
        out_shape=jax.ShapeDtypeStruct((m, n), jnp.float32),
        in_specs=[
            pl.BlockSpec(memory_space=pltpu.VMEM),
            pl.BlockSpec(memory_space=pltpu.VMEM),
        ],
        out_specs=pl.BlockSpec(memory_space=pltpu.VMEM),
        scratch_shapes=[
            pltpu.VMEM((N_DEV, sub, lane), jnp.float32),
            pltpu.SemaphoreType.DMA((N_DEV - 1,)),
            pltpu.SemaphoreType.DMA((N_DEV - 1,)),
        ],
        compiler_params=pltpu.CompilerParams(collective_id=0),
    )(x, gamma2d)


# baseline (device time: 29113 ns/iter reference)
import jax
import jax.numpy as jnp
from jax import lax
from jax.experimental import pallas as pl
from jax.experimental.pallas import tpu as pltpu

N_DEV = 4
EPS = 1e-5


def kernel(x, gamma):
    m, n = x.shape
    n_global = N_DEV * n
    gamma2d = gamma.reshape(1, n)

    def body(x_ref, g_ref, out_ref, comm_ref, send_sems, recv_sems):
        my_pos = lax.axis_index("i")
        left = jax.lax.rem(my_pos + N_DEV - 1, N_DEV)
        right = jax.lax.rem(my_pos + 1, N_DEV)

        barrier_sem = pltpu.get_barrier_semaphore()
        for nbr in [left, right]:
            pl.semaphore_signal(
                barrier_sem, inc=1,
                device_id=(nbr,), device_id_type=pl.DeviceIdType.MESH,
            )
        pl.semaphore_wait(barrier_sem, 2)

        xv = x_ref[:, :]
        comm_ref[0, :, :] = jnp.sum(xv * xv, axis=1, keepdims=True)

        for h in range(N_DEV - 1):
            rdma = pltpu.make_async_remote_copy(
                src_ref=comm_ref.at[h],
                dst_ref=comm_ref.at[h + 1],
                send_sem=send_sems.at[h],
                recv_sem=recv_sems.at[h],
                device_id=(right,),
                device_id_type=pl.DeviceIdType.MESH,
            )
            rdma.start()
            rdma.wait()

        total = (
            comm_ref[0, :, :] + comm_ref[1, :, :]
            + comm_ref[2, :, :] + comm_ref[3, :, :]
        )
        inv_rms = lax.rsqrt(total / n_global + EPS)
        out_ref[:, :] = xv * g_ref[0, :][None, :] * inv_rms

    return pl.pallas_call(
        body,
        out_shape=jax.ShapeDtypeStruct((m, n), jnp.float32),
        in_specs=[
            pl.BlockSpec(memory_space=pltpu.VMEM),
            pl.BlockSpec(memory_space=pltpu.VMEM),
        ],
        out_specs=pl.BlockSpec(memory_space=pltpu.VMEM),
        scratch_shapes=[
            pltpu.VMEM((N_DEV, m, 1), jnp.float32),
            pltpu.SemaphoreType.DMA((N_DEV - 1,)),
            pltpu.SemaphoreType.DMA((N_DEV - 1,)),
        ],
        compiler_params=pltpu.CompilerParams(collective_id=0),
    )(x, gamma2d)


# device time: 9041 ns/iter; 3.2201x vs baseline; 3.2201x over previous
import jax
import jax.numpy as jnp
from jax import lax
from jax.experimental import pallas as pl
from jax.experimental.pallas import tpu as pltpu

N_DEV = 4
EPS = 1e-5
SUB, LANE = 8, 128


def kernel(x, gamma):
    m, n = x.shape
    assert m == SUB * LANE
    n_global = N_DEV * n
    gamma2d = gamma.reshape(1, n)

    def body(x_ref, g_ref, out_ref, mine_ref, comm_ref, send_sems, recv_sems):
        my_pos = lax.axis_index("i")

        barrier_sem = pltpu.get_barrier_semaphore()
        for k in range(1, N_DEV):
            peer = lax.rem(my_pos + k, N_DEV)
            pl.semaphore_signal(
                barrier_sem, inc=1,
                device_id=(peer,), device_id_type=pl.DeviceIdType.MESH,
            )
        pl.semaphore_wait(barrier_sem, N_DEV - 1)

        xv = x_ref[:, :]
        partial = jnp.sum(xv * xv, axis=1)
        mine_ref[:, :] = partial.reshape(SUB, LANE)

        rdmas = []
        for k in range(1, N_DEV):
            dst = lax.rem(my_pos + k, N_DEV)
            rdma = pltpu.make_async_remote_copy(
                src_ref=mine_ref,
                dst_ref=comm_ref.at[k - 1],
                send_sem=send_sems.at[k - 1],
                recv_sem=recv_sems.at[k - 1],
                device_id=(dst,),
                device_id_type=pl.DeviceIdType.MESH,
            )
            rdma.start()
            rdmas.append(rdma)

        out_ref[:, :] = xv * g_ref[0, :][None, :]

        for rdma in rdmas:
            rdma.wait()

        total8 = (
            mine_ref[:, :] + comm_ref[0, :, :]
            + comm_ref[1, :, :] + comm_ref[2, :, :]
        )

        sub_i = lax.broadcasted_iota(jnp.int32, (LANE, LANE), 0)
        lane_i = lax.broadcasted_iota(jnp.int32, (LANE, LANE), 1)
        diag = sub_i == lane_i
        for b in range(SUB):
            vb = jnp.broadcast_to(total8[b : b + 1, :], (LANE, LANE))
            col = jnp.sum(jnp.where(diag, vb, 0.0), axis=1, keepdims=True)
            inv_rms = lax.rsqrt(col / n_global + EPS)
            blk = pl.ds(b * LANE, LANE)
            out_ref[blk, :] = out_ref[blk, :] * inv_rms

    return pl.pallas_call(
        body,
        out_shape=jax.ShapeDtypeStruct((m, n), jnp.float32),
        in_specs=[
            pl.BlockSpec(memory_space=pltpu.VMEM),
            pl.BlockSpec(memory_space=pltpu.VMEM),
        ],
        out_specs=pl.BlockSpec(memory_space=pltpu.VMEM),
        scratch_shapes=[
            pltpu.VMEM((SUB, LANE), jnp.float32),
            pltpu.VMEM((N_DEV - 1, SUB, LANE), jnp.float32),
            pltpu.SemaphoreType.DMA((N_DEV - 1,)),
            pltpu.SemaphoreType.DMA((N_DEV - 1,)),
        ],
        compiler_params=pltpu.CompilerParams(collective_id=0),
    )(x, gamma2d)


# device time: 8728 ns/iter; 3.3356x vs baseline; 1.0359x over previous
import jax
import jax.numpy as jnp
from jax import lax
from jax.experimental import pallas as pl
from jax.experimental.pallas import tpu as pltpu

N_DEV = 4
EPS = 1e-5
SUB, LANE = 8, 128


def kernel(x, gamma):
    m, n = x.shape
    assert m == SUB * LANE
    n_global = N_DEV * n
    gamma2d = gamma.reshape(1, n)

    def body(x_ref, g_ref, out_ref, mine_ref, comm_ref, send_sems, recv_sems):
        my_pos = lax.axis_index("i")

        barrier_sem = pltpu.get_barrier_semaphore()
        for k in range(1, N_DEV):
            peer = lax.rem(my_pos + k, N_DEV)
            pl.semaphore_signal(
                barrier_sem, inc=1,
                device_id=(peer,), device_id_type=pl.DeviceIdType.MESH,
            )

        xv = x_ref[:, :]
        partial = jnp.sum(xv * xv, axis=1)
        mine_ref[:, :] = partial.reshape(SUB, LANE)

        pl.semaphore_wait(barrier_sem, N_DEV - 1)

        rdmas = []
        for k in range(1, N_DEV):
            dst = lax.rem(my_pos + k, N_DEV)
            rdma = pltpu.make_async_remote_copy(
                src_ref=mine_ref,
                dst_ref=comm_ref.at[k - 1],
                send_sem=send_sems.at[k - 1],
                recv_sem=recv_sems.at[k - 1],
                device_id=(dst,),
                device_id_type=pl.DeviceIdType.MESH,
            )
            rdma.start()
            rdmas.append(rdma)

        out_ref[:, :] = xv * g_ref[0, :][None, :]

        for rdma in rdmas:
            rdma.wait()

        total8 = (
            mine_ref[:, :] + comm_ref[0, :, :]
            + comm_ref[1, :, :] + comm_ref[2, :, :]
        )

        sub_i = lax.broadcasted_iota(jnp.int32, (LANE, LANE), 0)
        lane_i = lax.broadcasted_iota(jnp.int32, (LANE, LANE), 1)
        diag = sub_i == lane_i
        for b in range(SUB):
            vb = jnp.broadcast_to(total8[b : b + 1, :], (LANE, LANE))
            col = jnp.sum(jnp.where(diag, vb, 0.0), axis=1, keepdims=True)
            inv_rms = lax.rsqrt(col / n_global + EPS)
            blk = pl.ds(b * LANE, LANE)
            out_ref[blk, :] = out_ref[blk, :] * inv_rms

    return pl.pallas_call(
        body,
        out_shape=jax.ShapeDtypeStruct((m, n), jnp.float32),
        in_specs=[
            pl.BlockSpec(memory_space=pltpu.VMEM),
            pl.BlockSpec(memory_space=pltpu.VMEM),
        ],
        out_specs=pl.BlockSpec(memory_space=pltpu.VMEM),
        scratch_shapes=[
            pltpu.VMEM((SUB, LANE), jnp.float32),
            pltpu.VMEM((N_DEV - 1, SUB, LANE), jnp.float32),
            pltpu.SemaphoreType.DMA((N_DEV - 1,)),
            pltpu.SemaphoreType.DMA((N_DEV - 1,)),
        ],
        compiler_params=pltpu.CompilerParams(collective_id=0),
    )(x, gamma2d)
